# baseline (speedup 1.0000x reference)
import jax
import jax.numpy as jnp
from jax.experimental import pallas as pl
from jax.experimental.pallas import tpu as pltpu

_BLK = 1024
_NUM_USERS = 16


def _fused_kernel(idx_ref, his_ref, q_ref, out_ref, acc_ref, m_ref, s_ref):
    i = pl.program_id(0)
    nblk = pl.num_programs(0)

    @pl.when(i == 0)
    def _init():
        acc_ref[:] = jnp.zeros_like(acc_ref)
        m_ref[:] = jnp.full_like(m_ref, -jnp.inf)
        s_ref[:] = jnp.zeros_like(s_ref)

    h = his_ref[:]
    q = q_ref[:]
    hb = h.astype(jnp.bfloat16).astype(jnp.float32)
    qb = q.astype(jnp.bfloat16).astype(jnp.float32)
    scores = jnp.sum(hb * qb, axis=1).reshape(1, _BLK)

    pos = jax.lax.broadcasted_iota(jnp.int32, (1, _BLK), 1) + i * _BLK
    starts = jnp.stack([idx_ref[u] for u in range(_NUM_USERS)]).reshape(
        _NUM_USERS, 1)
    ends = jnp.stack([idx_ref[u + 1] for u in range(_NUM_USERS)]).reshape(
        _NUM_USERS, 1)
    mask = (pos >= starts) & (pos < ends)

    neg_inf = jnp.float32(-jnp.inf)
    masked = jnp.where(mask, scores, neg_inf)
    m_blk = jnp.max(masked, axis=1, keepdims=True)
    m_old = m_ref[:]
    m_new = jnp.maximum(m_old, m_blk)
    alpha = jnp.where(m_new == neg_inf, 1.0, jnp.exp(m_old - m_new))
    e = jnp.where(mask, jnp.exp(scores - m_new), 0.0)
    s_ref[:] = s_ref[:] * alpha + jnp.sum(e, axis=1, keepdims=True)
    acc_ref[:] = acc_ref[:] * alpha + jax.lax.dot_general(
        e, h, (((1,), (0,)), ((), ())),
        preferred_element_type=jnp.float32,
        precision=jax.lax.Precision.HIGHEST)
    m_ref[:] = m_new

    @pl.when(i == nblk - 1)
    def _fin():
        out_ref[:] = acc_ref[:] / s_ref[:]


@jax.jit
def kernel(his_embs, user_indices, query_vector):
    total, dim = his_embs.shape
    nblk = total // _BLK
    q2 = query_vector.reshape(1, dim)
    grid_spec = pltpu.PrefetchScalarGridSpec(
        num_scalar_prefetch=1,
        grid=(nblk,),
        in_specs=[
            pl.BlockSpec((_BLK, dim), lambda i, idx: (i, 0)),
            pl.BlockSpec((1, dim), lambda i, idx: (0, 0)),
        ],
        out_specs=pl.BlockSpec((_NUM_USERS, dim), lambda i, idx: (0, 0)),
        scratch_shapes=[
            pltpu.VMEM((_NUM_USERS, dim), jnp.float32),
            pltpu.VMEM((_NUM_USERS, 1), jnp.float32),
            pltpu.VMEM((_NUM_USERS, 1), jnp.float32),
        ],
    )
    return pl.pallas_call(
        _fused_kernel,
        grid_spec=grid_spec,
        out_shape=jax.ShapeDtypeStruct((_NUM_USERS, dim), jnp.float32),
        compiler_params=pltpu.CompilerParams(
            dimension_semantics=("arbitrary",)),
    )(user_indices.astype(jnp.int32), his_embs, q2)


# speedup vs baseline: 1.1257x; 1.1257x over previous
import jax
import jax.numpy as jnp
from jax.experimental import pallas as pl
from jax.experimental.pallas import tpu as pltpu

_BLK = 1024
_NUM_USERS = 16


def _fused_kernel(idx_ref, his_ref, q_ref, out_ref, acc_ref, m_ref, s_ref):
    i = pl.program_id(0)
    nblk = pl.num_programs(0)

    @pl.when(i == 0)
    def _init():
        acc_ref[:] = jnp.zeros_like(acc_ref)
        m_ref[:] = jnp.full_like(m_ref, -jnp.inf)
        s_ref[:] = jnp.zeros_like(s_ref)

    h = his_ref[:]
    q = q_ref[:]
    h_hi = h.astype(jnp.bfloat16)
    hb = h_hi.astype(jnp.float32)
    qb = q.astype(jnp.bfloat16).astype(jnp.float32)
    scores = jnp.sum(hb * qb, axis=1).reshape(1, _BLK)

    pos = jax.lax.broadcasted_iota(jnp.int32, (1, _BLK), 1) + i * _BLK
    starts = jnp.stack([idx_ref[u] for u in range(_NUM_USERS)]).reshape(
        _NUM_USERS, 1)
    ends = jnp.stack([idx_ref[u + 1] for u in range(_NUM_USERS)]).reshape(
        _NUM_USERS, 1)
    mask = (pos >= starts) & (pos < ends)

    neg_inf = jnp.float32(-jnp.inf)
    masked = jnp.where(mask, scores, neg_inf)
    m_blk = jnp.max(masked, axis=1, keepdims=True)
    m_old = m_ref[:]
    m_new = jnp.maximum(m_old, m_blk)
    alpha = jnp.where(m_new == neg_inf, 1.0, jnp.exp(m_old - m_new))
    e = jnp.where(mask, jnp.exp(scores - m_new), 0.0)
    s_ref[:] = s_ref[:] * alpha + jnp.sum(e, axis=1, keepdims=True)
    h_lo = (h - hb).astype(jnp.bfloat16)
    e_hi = e.astype(jnp.bfloat16)
    e_lo = (e - e_hi.astype(jnp.float32)).astype(jnp.bfloat16)

    def _dot(a, b):
        return jax.lax.dot_general(
            a, b, (((1,), (0,)), ((), ())),
            preferred_element_type=jnp.float32)

    pooled = _dot(e_hi, h_lo) + _dot(e_lo, h_hi) + _dot(e_hi, h_hi)
    acc_ref[:] = acc_ref[:] * alpha + pooled
    m_ref[:] = m_new

    @pl.when(i == nblk - 1)
    def _fin():
        out_ref[:] = acc_ref[:] / s_ref[:]


@jax.jit
def kernel(his_embs, user_indices, query_vector):
    total, dim = his_embs.shape
    nblk = total // _BLK
    q2 = query_vector.reshape(1, dim)
    grid_spec = pltpu.PrefetchScalarGridSpec(
        num_scalar_prefetch=1,
        grid=(nblk,),
        in_specs=[
            pl.BlockSpec((_BLK, dim), lambda i, idx: (i, 0)),
            pl.BlockSpec((1, dim), lambda i, idx: (0, 0)),
        ],
        out_specs=pl.BlockSpec((_NUM_USERS, dim), lambda i, idx: (0, 0)),
        scratch_shapes=[
            pltpu.VMEM((_NUM_USERS, dim), jnp.float32),
            pltpu.VMEM((_NUM_USERS, 1), jnp.float32),
            pltpu.VMEM((_NUM_USERS, 1), jnp.float32),
        ],
    )
    return pl.pallas_call(
        _fused_kernel,
        grid_spec=grid_spec,
        out_shape=jax.ShapeDtypeStruct((_NUM_USERS, dim), jnp.float32),
        compiler_params=pltpu.CompilerParams(
            dimension_semantics=("arbitrary",)),
    )(user_indices.astype(jnp.int32), his_embs, q2)


# speedup vs baseline: 2.1214x; 1.8844x over previous
import jax
import jax.numpy as jnp
from jax.experimental import pallas as pl
from jax.experimental.pallas import tpu as pltpu

_BLK = 1024
_NUM_USERS = 16


def _fused_kernel(idx_ref, his_ref, q_ref, out_ref, acc_ref, m_ref, s_ref):
    i = pl.program_id(0)
    nblk = pl.num_programs(0)

    @pl.when(i == 0)
    def _init():
        acc_ref[:] = jnp.zeros_like(acc_ref)
        m_ref[:] = jnp.full_like(m_ref, -jnp.inf)
        s_ref[:] = jnp.zeros_like(s_ref)

    h = his_ref[:]
    q = q_ref[:]
    h_hi = h.astype(jnp.bfloat16)
    hb = h_hi.astype(jnp.float32)
    qb = q.astype(jnp.bfloat16).astype(jnp.float32)
    scores = jnp.sum(hb * qb, axis=1).reshape(1, _BLK)

    pos = jax.lax.broadcasted_iota(jnp.int32, (1, _BLK), 1) + i * _BLK
    starts = jnp.stack([idx_ref[u] for u in range(_NUM_USERS)]).reshape(
        _NUM_USERS, 1)
    ends = jnp.stack([idx_ref[u + 1] for u in range(_NUM_USERS)]).reshape(
        _NUM_USERS, 1)
    mask = (pos >= starts) & (pos < ends)

    neg_inf = jnp.float32(-jnp.inf)
    masked = jnp.where(mask, scores, neg_inf)
    m_blk = jnp.max(masked, axis=1, keepdims=True)
    m_old = m_ref[:]
    m_new = jnp.maximum(m_old, m_blk)
    alpha = jnp.where(m_new == neg_inf, 1.0, jnp.exp(m_old - m_new))
    ownf = jnp.sum(mask.astype(jnp.float32), axis=0, keepdims=True)
    shift = (jnp.sum(jnp.where(mask, m_new, 0.0), axis=0, keepdims=True)
             + (1.0 - ownf) * scores)
    ev = jnp.exp(scores - shift)
    e = jnp.where(mask, ev, 0.0)
    s_ref[:] = s_ref[:] * alpha + jnp.sum(e, axis=1, keepdims=True)
    h_lo = (h - hb).astype(jnp.bfloat16)
    e_hi = e.astype(jnp.bfloat16)
    e_lo = (e - e_hi.astype(jnp.float32)).astype(jnp.bfloat16)

    def _dot(a, b):
        return jax.lax.dot_general(
            a, b, (((1,), (0,)), ((), ())),
            preferred_element_type=jnp.float32)

    pooled = _dot(e_hi, h_lo) + _dot(e_lo, h_hi) + _dot(e_hi, h_hi)
    acc_ref[:] = acc_ref[:] * alpha + pooled
    m_ref[:] = m_new

    @pl.when(i == nblk - 1)
    def _fin():
        out_ref[:] = acc_ref[:] / s_ref[:]


@jax.jit
def kernel(his_embs, user_indices, query_vector):
    total, dim = his_embs.shape
    nblk = total // _BLK
    q2 = query_vector.reshape(1, dim)
    grid_spec = pltpu.PrefetchScalarGridSpec(
        num_scalar_prefetch=1,
        grid=(nblk,),
        in_specs=[
            pl.BlockSpec((_BLK, dim), lambda i, idx: (i, 0)),
            pl.BlockSpec((1, dim), lambda i, idx: (0, 0)),
        ],
        out_specs=pl.BlockSpec((_NUM_USERS, dim), lambda i, idx: (0, 0)),
        scratch_shapes=[
            pltpu.VMEM((_NUM_USERS, dim), jnp.float32),
            pltpu.VMEM((_NUM_USERS, 1), jnp.float32),
            pltpu.VMEM((_NUM_USERS, 1), jnp.float32),
        ],
    )
    return pl.pallas_call(
        _fused_kernel,
        grid_spec=grid_spec,
        out_shape=jax.ShapeDtypeStruct((_NUM_USERS, dim), jnp.float32),
        compiler_params=pltpu.CompilerParams(
            dimension_semantics=("arbitrary",)),
    )(user_indices.astype(jnp.int32), his_embs, q2)


# speedup vs baseline: 2.4248x; 1.1430x over previous
import jax
import jax.numpy as jnp
from jax.experimental import pallas as pl
from jax.experimental.pallas import tpu as pltpu

_BLK = 2048
_NUM_USERS = 16


def _fused_kernel(idx_ref, his_ref, q_ref, out_ref, acc_ref, m_ref, s_ref):
    i = pl.program_id(0)
    nblk = pl.num_programs(0)

    @pl.when(i == 0)
    def _init():
        acc_ref[:] = jnp.zeros_like(acc_ref)
        m_ref[:] = jnp.full_like(m_ref, -jnp.inf)
        s_ref[:] = jnp.zeros_like(s_ref)

    h = his_ref[:]
    q = q_ref[:]
    h_hi = h.astype(jnp.bfloat16)
    hb = h_hi.astype(jnp.float32)
    qb = q.astype(jnp.bfloat16).astype(jnp.float32)
    scores = jnp.sum(hb * qb, axis=1).reshape(1, _BLK)

    pos = jax.lax.broadcasted_iota(jnp.int32, (1, _BLK), 1) + i * _BLK
    starts = jnp.stack([idx_ref[u] for u in range(_NUM_USERS)]).reshape(
        _NUM_USERS, 1)
    ends = jnp.stack([idx_ref[u + 1] for u in range(_NUM_USERS)]).reshape(
        _NUM_USERS, 1)
    mask = (pos >= starts) & (pos < ends)

    neg_inf = jnp.float32(-jnp.inf)
    masked = jnp.where(mask, scores, neg_inf)
    m_blk = jnp.max(masked, axis=1, keepdims=True)
    m_old = m_ref[:]
    m_new = jnp.maximum(m_old, m_blk)
    alpha = jnp.where(m_new == neg_inf, 1.0, jnp.exp(m_old - m_new))
    ownf = jnp.sum(mask.astype(jnp.float32), axis=0, keepdims=True)
    shift = (jnp.sum(jnp.where(mask, m_new, 0.0), axis=0, keepdims=True)
             + (1.0 - ownf) * scores)
    ev = jnp.exp(scores - shift)
    e = jnp.where(mask, ev, 0.0)
    s_ref[:] = s_ref[:] * alpha + jnp.sum(e, axis=1, keepdims=True)
    h_lo = (h - hb).astype(jnp.bfloat16)
    e_hi = e.astype(jnp.bfloat16)
    e_lo = (e - e_hi.astype(jnp.float32)).astype(jnp.bfloat16)

    def _dot(a, b):
        return jax.lax.dot_general(
            a, b, (((1,), (0,)), ((), ())),
            preferred_element_type=jnp.float32)

    pooled = _dot(e_hi, h_lo) + _dot(e_lo, h_hi) + _dot(e_hi, h_hi)
    acc_ref[:] = acc_ref[:] * alpha + pooled
    m_ref[:] = m_new

    @pl.when(i == nblk - 1)
    def _fin():
        out_ref[:] = acc_ref[:] / s_ref[:]


@jax.jit
def kernel(his_embs, user_indices, query_vector):
    total, dim = his_embs.shape
    nblk = total // _BLK
    q2 = query_vector.reshape(1, dim)
    grid_spec = pltpu.PrefetchScalarGridSpec(
        num_scalar_prefetch=1,
        grid=(nblk,),
        in_specs=[
            pl.BlockSpec((_BLK, dim), lambda i, idx: (i, 0)),
            pl.BlockSpec((1, dim), lambda i, idx: (0, 0)),
        ],
        out_specs=pl.BlockSpec((_NUM_USERS, dim), lambda i, idx: (0, 0)),
        scratch_shapes=[
            pltpu.VMEM((_NUM_USERS, dim), jnp.float32),
            pltpu.VMEM((_NUM_USERS, 1), jnp.float32),
            pltpu.VMEM((_NUM_USERS, 1), jnp.float32),
        ],
    )
    return pl.pallas_call(
        _fused_kernel,
        grid_spec=grid_spec,
        out_shape=jax.ShapeDtypeStruct((_NUM_USERS, dim), jnp.float32),
        compiler_params=pltpu.CompilerParams(
            dimension_semantics=("arbitrary",)),
    )(user_indices.astype(jnp.int32), his_embs, q2)


# speedup vs baseline: 2.4737x; 1.0202x over previous
import jax
import jax.numpy as jnp
from jax.experimental import pallas as pl
from jax.experimental.pallas import tpu as pltpu

_BLK = 4096
_NUM_USERS = 16


def _fused_kernel(idx_ref, his_ref, q_ref, out_ref, acc_ref, m_ref, s_ref):
    i = pl.program_id(0)
    nblk = pl.num_programs(0)

    @pl.when(i == 0)
    def _init():
        acc_ref[:] = jnp.zeros_like(acc_ref)
        m_ref[:] = jnp.full_like(m_ref, -jnp.inf)
        s_ref[:] = jnp.zeros_like(s_ref)

    h = his_ref[:]
    q = q_ref[:]
    h_hi = h.astype(jnp.bfloat16)
    hb = h_hi.astype(jnp.float32)
    qb = q.astype(jnp.bfloat16).astype(jnp.float32)
    scores = jnp.sum(hb * qb, axis=1).reshape(1, _BLK)

    pos = jax.lax.broadcasted_iota(jnp.int32, (1, _BLK), 1) + i * _BLK
    starts = jnp.stack([idx_ref[u] for u in range(_NUM_USERS)]).reshape(
        _NUM_USERS, 1)
    ends = jnp.stack([idx_ref[u + 1] for u in range(_NUM_USERS)]).reshape(
        _NUM_USERS, 1)
    mask = (pos >= starts) & (pos < ends)

    neg_inf = jnp.float32(-jnp.inf)
    masked = jnp.where(mask, scores, neg_inf)
    m_blk = jnp.max(masked, axis=1, keepdims=True)
    m_old = m_ref[:]
    m_new = jnp.maximum(m_old, m_blk)
    alpha = jnp.where(m_new == neg_inf, 1.0, jnp.exp(m_old - m_new))
    ownf = jnp.sum(mask.astype(jnp.float32), axis=0, keepdims=True)
    shift = (jnp.sum(jnp.where(mask, m_new, 0.0), axis=0, keepdims=True)
             + (1.0 - ownf) * scores)
    ev = jnp.exp(scores - shift)
    e = jnp.where(mask, ev, 0.0)
    s_ref[:] = s_ref[:] * alpha + jnp.sum(e, axis=1, keepdims=True)
    h_lo = (h - hb).astype(jnp.bfloat16)
    e_hi = e.astype(jnp.bfloat16)
    e_lo = (e - e_hi.astype(jnp.float32)).astype(jnp.bfloat16)

    def _dot(a, b):
        return jax.lax.dot_general(
            a, b, (((1,), (0,)), ((), ())),
            preferred_element_type=jnp.float32)

    pooled = _dot(e_hi, h_lo) + _dot(e_lo, h_hi) + _dot(e_hi, h_hi)
    acc_ref[:] = acc_ref[:] * alpha + pooled
    m_ref[:] = m_new

    @pl.when(i == nblk - 1)
    def _fin():
        out_ref[:] = acc_ref[:] / s_ref[:]


@jax.jit
def kernel(his_embs, user_indices, query_vector):
    total, dim = his_embs.shape
    nblk = total // _BLK
    q2 = query_vector.reshape(1, dim)
    grid_spec = pltpu.PrefetchScalarGridSpec(
        num_scalar_prefetch=1,
        grid=(nblk,),
        in_specs=[
            pl.BlockSpec((_BLK, dim), lambda i, idx: (i, 0)),
            pl.BlockSpec((1, dim), lambda i, idx: (0, 0)),
        ],
        out_specs=pl.BlockSpec((_NUM_USERS, dim), lambda i, idx: (0, 0)),
        scratch_shapes=[
            pltpu.VMEM((_NUM_USERS, dim), jnp.float32),
            pltpu.VMEM((_NUM_USERS, 1), jnp.float32),
            pltpu.VMEM((_NUM_USERS, 1), jnp.float32),
        ],
    )
    return pl.pallas_call(
        _fused_kernel,
        grid_spec=grid_spec,
        out_shape=jax.ShapeDtypeStruct((_NUM_USERS, dim), jnp.float32),
        compiler_params=pltpu.CompilerParams(
            dimension_semantics=("arbitrary",)),
    )(user_indices.astype(jnp.int32), his_embs, q2)
